# baseline (speedup 1.0000x reference)
import functools

import jax
import jax.numpy as jnp
from jax import lax
from jax.experimental import pallas as pl
from jax.experimental.pallas import tpu as pltpu
from jax.experimental.pallas import tpu_sc as plsc

N = 10000
E = 320000
G = 64
D = 128
H = 128
U = 32
A = 16

NP = 10240
NC = 2
NS = 16
DH = D // 2
CH = 120
CPT = 168
EP = CPT * CH * NS
CPD = CPT // 2
RPT = NP // NS
R = 1024
NB = NP // R

_f32 = jnp.float32


def _mesh():
    return plsc.VectorSubcoreMesh(
        core_axis_name="c", subcore_axis_name="s", num_cores=NC, num_subcores=NS
    )


def _deg_body(sd_hbm, ones_hbm, zeros_hbm, out_hbm, idxall, onesv, acc, sem):
    c = lax.axis_index("c")
    s = lax.axis_index("s")
    pltpu.sync_copy(zeros_hbm, acc.at[pl.ds(s * RPT, RPT)])
    pltpu.sync_copy(ones_hbm, onesv)
    pltpu.sync_copy(sd_hbm.at[s, pl.ds(c * CPD, CPD)], idxall)
    plsc.subcore_barrier()

    def fire(g, carry):
        pltpu.make_async_copy(onesv, acc.at[idxall.at[g, 1]], sem).start(add=True)
        return carry

    def drain(g, carry):
        pltpu.make_async_copy(onesv, acc.at[idxall.at[g, 1]], sem).wait()
        return carry

    lax.fori_loop(0, CPD, fire, 0)
    lax.fori_loop(0, CPD, drain, 0)
    plsc.subcore_barrier()
    pltpu.sync_copy(acc.at[pl.ds(s * RPT, RPT)], out_hbm.at[c, pl.ds(s * RPT, RPT)])


_deg_call = pl.kernel(
    _deg_body,
    out_type=jax.ShapeDtypeStruct((NC, NP), _f32),
    mesh=_mesh(),
    scratch_types=[
        pltpu.VMEM((CPD, 2, CH), jnp.int32),
        pltpu.VMEM((CH,), _f32),
        pltpu.VMEM_SHARED((NP,), _f32),
        pltpu.SemaphoreType.DMA,
    ],
)


NBUF = 6
NIDX = 8
LEAD = 3


def _edge_body(hw_hbm, sd_hbm, zrow_hbm, out_hbm, idxb, stage, table, acc,
               isem, gsem, ssem):
    c = lax.axis_index("c")
    s = lax.axis_index("s")
    cols = pl.ds(c * DH, DH)

    def idx_load(g, slot):
        return pltpu.make_async_copy(sd_hbm.at[s, g], idxb.at[slot], isem.at[slot])

    def gath(slot, b):
        return pltpu.make_async_copy(table.at[idxb.at[slot, 0]], stage.at[b],
                                     gsem.at[b])

    def scat(slot, b):
        return pltpu.make_async_copy(stage.at[b], acc.at[idxb.at[slot, 1]],
                                     ssem.at[b])

    for j in range(NIDX - 1):
        idx_load(j, j).start()
    rows = pl.ds(s * RPT, RPT)
    pltpu.sync_copy(hw_hbm.at[rows, cols], table.at[rows])
    for r in range(RPT // 128):
        pltpu.sync_copy(zrow_hbm, acc.at[pl.ds(s * RPT + r * 128, 128)])
    plsc.subcore_barrier()
    for j in range(LEAD):
        idx_load(j, j).wait()
        gath(j, j).start()

    def body(i, carry):
        b = lax.rem(i, NBUF)
        sl = jnp.bitwise_and(i, NIDX - 1)
        f = i + LEAD
        slf = jnp.bitwise_and(f, NIDX - 1)
        bf = lax.rem(f, NBUF)

        @pl.when(f < CPT)
        def _():
            idx_load(f, slf).wait()
            fb = f - NBUF

            @pl.when(fb >= 0)
            def _():
                scat(jnp.bitwise_and(fb, NIDX - 1), bf).wait()

            gath(slf, bf).start()
            nxt = fb + NIDX

            @pl.when((nxt >= NIDX - 1) & (nxt < CPT))
            def _():
                idx_load(nxt, jnp.bitwise_and(nxt, NIDX - 1)).start()

        gath(sl, b).wait()
        scat(sl, b).start(add=True)
        return carry

    lax.fori_loop(0, CPT, body, 0)
    for g in range(CPT - NBUF, CPT):
        scat(g & (NIDX - 1), g % NBUF).wait()
    plsc.subcore_barrier()
    pltpu.sync_copy(acc.at[rows], out_hbm.at[rows, cols])


_edge_call = pl.kernel(
    _edge_body,
    out_type=jax.ShapeDtypeStruct((NP, D), _f32),
    mesh=_mesh(),
    compiler_params=pltpu.CompilerParams(use_tc_tiling_on_sc=False),
    scratch_types=[
        pltpu.VMEM((NIDX, 2, CH), jnp.int32),
        pltpu.VMEM((NBUF, CH, DH), _f32),
        pltpu.VMEM_SHARED((NP, DH), _f32),
        pltpu.VMEM_SHARED((NP, DH), _f32),
        pltpu.SemaphoreType.DMA((NIDX,)),
        pltpu.SemaphoreType.DMA((NBUF,)),
        pltpu.SemaphoreType.DMA((NBUF,)),
    ],
)


def _mm_raw_body(x_ref, w_ref, o_ref):
    o_ref[...] = jnp.dot(x_ref[...], w_ref[...], preferred_element_type=_f32)


def _mm_raw(xp, w1):
    return pl.pallas_call(
        _mm_raw_body,
        grid=(NB,),
        in_specs=[
            pl.BlockSpec((R, D), lambda i: (i, 0)),
            pl.BlockSpec((D, H), lambda i: (0, 0)),
        ],
        out_specs=pl.BlockSpec((R, H), lambda i: (i, 0)),
        out_shape=jax.ShapeDtypeStruct((NP, H), _f32),
    )(xp, w1)


def _scale_body(xw_ref, d_ref, o_ref, t_ref):
    dinv = lax.rsqrt(d_ref[0:1, :] + d_ref[1:2, :] + 1.0)
    t = jnp.transpose(dinv, (1, 0))
    t_ref[...] = jnp.broadcast_to(t, (R, D))
    o_ref[...] = xw_ref[...] * t


def _mm_scale(xw, degs):
    return pl.pallas_call(
        _scale_body,
        grid=(NB,),
        in_specs=[
            pl.BlockSpec((R, D), lambda i: (i, 0)),
            pl.BlockSpec((NC, R), lambda i: (0, i)),
        ],
        out_specs=[
            pl.BlockSpec((R, H), lambda i: (i, 0)),
            pl.BlockSpec((R, D), lambda i: (i, 0)),
        ],
        out_shape=[
            jax.ShapeDtypeStruct((NP, H), _f32),
            jax.ShapeDtypeStruct((NP, D), _f32),
        ],
    )(xw, degs)


def _layer_body(p_ref, hw_ref, t_ref, w_ref, b_ref, o_ref):
    t = t_ref[...]
    h = jnp.maximum((p_ref[...] + hw_ref[...]) * t + b_ref[...], 0.0)
    o_ref[...] = jnp.dot(h, w_ref[...], preferred_element_type=_f32) * t


def _layer_fuse(parts, hw, tinv, w2, b1):
    return pl.pallas_call(
        _layer_body,
        grid=(NB,),
        in_specs=[
            pl.BlockSpec((R, D), lambda i: (i, 0)),
            pl.BlockSpec((R, D), lambda i: (i, 0)),
            pl.BlockSpec((R, D), lambda i: (i, 0)),
            pl.BlockSpec((D, H), lambda i: (0, 0)),
            pl.BlockSpec((1, H), lambda i: (0, 0)),
        ],
        out_specs=pl.BlockSpec((R, H), lambda i: (i, 0)),
        out_shape=jax.ShapeDtypeStruct((NP, H), _f32),
    )(parts, hw, tinv, w2, b1)


def _pool_body(p_ref, hw_ref, t_ref, b2_ref, bt_ref, u_ref,
               aw1h_ref, aw1u_ref, ab1_ref, aw2_ref, ab2_ref,
               cw1h_ref, cw1u_ref, cb1_ref, cw2_ref, cb2_ref,
               oa_ref, ov_ref, pacc, cacc):
    i = pl.program_id(0)

    @pl.when(i == 0)
    def _init():
        pacc[...] = jnp.zeros_like(pacc)
        cacc[...] = jnp.zeros_like(cacc)

    h2 = jnp.maximum((p_ref[...] + hw_ref[...]) * t_ref[...] + b2_ref[...], 0.0)
    onehot = (bt_ref[...] == lax.broadcasted_iota(jnp.int32, (G, R), 0)).astype(_f32)
    pacc[...] += jnp.dot(onehot, h2, preferred_element_type=_f32)
    cacc[...] += jnp.dot(onehot, jnp.ones((R, D), _f32), preferred_element_type=_f32)

    @pl.when(i == NB - 1)
    def _heads():
        pooled = pacc[...] / jnp.maximum(cacc[...], 1.0)
        ah = jnp.maximum(
            jnp.dot(pooled, aw1h_ref[...], preferred_element_type=_f32)
            + jnp.dot(u_ref[...], aw1u_ref[...], preferred_element_type=_f32)
            + ab1_ref[...], 0.0)
        oa_ref[...] = jnp.dot(ah, aw2_ref[...], preferred_element_type=_f32) + ab2_ref[...]
        ch = jnp.maximum(
            jnp.dot(pooled, cw1h_ref[...], preferred_element_type=_f32)
            + jnp.dot(u_ref[...], cw1u_ref[...], preferred_element_type=_f32)
            + cb1_ref[...], 0.0)
        ov_ref[...] = jnp.dot(ch, cw2_ref[...], preferred_element_type=_f32) + cb2_ref[...]


def _pool_heads(parts, hw, tinv, b2, bt, u, aW1, ab1, aW2, ab2,
                cW1, cb1, cW2, cb2):
    full = lambda shape: pl.BlockSpec(shape, lambda i: tuple(0 for _ in shape))
    return pl.pallas_call(
        _pool_body,
        grid=(NB,),
        in_specs=[
            pl.BlockSpec((R, D), lambda i: (i, 0)),
            pl.BlockSpec((R, D), lambda i: (i, 0)),
            pl.BlockSpec((R, D), lambda i: (i, 0)),
            full((1, H)),
            pl.BlockSpec((1, R), lambda i: (0, i)),
            full((G, U)),
            full((H, H)), full((U, H)), full((1, H)),
            full((H, A)), full((1, A)),
            full((H, H)), full((U, H)), full((1, H)),
            full((H, 1)), full((1, 1)),
        ],
        out_specs=[
            pl.BlockSpec((G, A), lambda i: (0, 0)),
            pl.BlockSpec((G, 1), lambda i: (0, 0)),
        ],
        out_shape=[
            jax.ShapeDtypeStruct((G, A), _f32),
            jax.ShapeDtypeStruct((G, 1), _f32),
        ],
        scratch_shapes=[
            pltpu.VMEM((G, H), _f32),
            pltpu.VMEM((G, H), _f32),
        ],
    )(parts, hw, tinv, b2, bt, u,
      aW1[:H], aW1[H:], ab1, aW2, ab2,
      cW1[:H], cW1[H:], cb1, cW2, cb2)


def kernel(x, u, W1, b1, W2, b2, aW1, ab1, aW2, ab2, cW1, cb1, cW2, cb2,
           edge_index, batch):
    xp = jnp.zeros((NP, D), _f32).at[:N].set(x)
    pad = jnp.full((2, EP - E), N, jnp.int32)
    srcdst = (jnp.concatenate([edge_index, pad], axis=1)
              .reshape(2, NS, CPT, CH).transpose(1, 2, 0, 3))
    btp = jnp.concatenate([batch, jnp.full((NP - N,), G, jnp.int32)]).reshape(1, NP)

    ones_row = jnp.ones((CH,), _f32)
    zeros_seg = jnp.zeros((RPT,), _f32)
    zeros_row = jnp.zeros((128, DH), _f32)

    degs = _deg_call(srcdst, ones_row, zeros_seg)

    xw = _mm_raw(xp, W1)
    hw1, tinv = _mm_scale(xw, degs)
    parts1 = _edge_call(hw1, srcdst, zeros_row)
    hw2 = _layer_fuse(parts1, hw1, tinv, W2, b1.reshape(1, H))
    parts2 = _edge_call(hw2, srcdst, zeros_row)

    oa, ov = _pool_heads(
        parts2, hw2, tinv, b2.reshape(1, H), btp, u,
        aW1, ab1.reshape(1, H), aW2, ab2.reshape(1, A),
        cW1, cb1.reshape(1, H), cW2, cb2.reshape(1, 1))
    return (oa, ov)


# speedup vs baseline: 1.0736x; 1.0736x over previous
import functools

import jax
import jax.numpy as jnp
from jax import lax
from jax.experimental import pallas as pl
from jax.experimental.pallas import tpu as pltpu
from jax.experimental.pallas import tpu_sc as plsc

N = 10000
E = 320000
G = 64
D = 128
H = 128
U = 32
A = 16

NP = 10240
NC = 2
NS = 16
DH = D // 2
CH = 128
CPT = 158
EP = CPT * CH * NS
CPD = CPT // 2
RPT = NP // NS
R = 1024
NB = NP // R

_f32 = jnp.float32


def _mesh():
    return plsc.VectorSubcoreMesh(
        core_axis_name="c", subcore_axis_name="s", num_cores=NC, num_subcores=NS
    )


def _deg_body(sd_hbm, ones_hbm, zeros_hbm, out_hbm, idxall, onesv, acc, sem):
    c = lax.axis_index("c")
    s = lax.axis_index("s")
    pltpu.sync_copy(zeros_hbm, acc.at[pl.ds(s * RPT, RPT)])
    pltpu.sync_copy(ones_hbm, onesv)
    pltpu.sync_copy(sd_hbm.at[s, pl.ds(c * CPD, CPD)], idxall)
    plsc.subcore_barrier()

    def fire(g, carry):
        pltpu.make_async_copy(onesv, acc.at[idxall.at[g, 1]], sem).start(add=True)
        return carry

    def drain(g, carry):
        pltpu.make_async_copy(onesv, acc.at[idxall.at[g, 1]], sem).wait()
        return carry

    lax.fori_loop(0, CPD, fire, 0)
    lax.fori_loop(0, CPD, drain, 0)
    plsc.subcore_barrier()
    pltpu.sync_copy(acc.at[pl.ds(s * RPT, RPT)], out_hbm.at[c, pl.ds(s * RPT, RPT)])


_deg_call = pl.kernel(
    _deg_body,
    out_type=jax.ShapeDtypeStruct((NC, NP), _f32),
    mesh=_mesh(),
    scratch_types=[
        pltpu.VMEM((CPD, 2, CH), jnp.int32),
        pltpu.VMEM((CH,), _f32),
        pltpu.VMEM_SHARED((NP,), _f32),
        pltpu.SemaphoreType.DMA,
    ],
)


NBUF = 5
NIDX = 8
LEAD = 3


def _edge_body(hw_hbm, sd_hbm, zrow_hbm, out_hbm, idxb, stage, table, acc,
               isem, gsem, ssem):
    c = lax.axis_index("c")
    s = lax.axis_index("s")
    cols = pl.ds(c * DH, DH)

    def idx_load(g, slot):
        return pltpu.make_async_copy(sd_hbm.at[s, g], idxb.at[slot], isem.at[slot])

    def gath(slot, b):
        return pltpu.make_async_copy(table.at[idxb.at[slot, 0]], stage.at[b],
                                     gsem.at[b])

    def scat(slot, b):
        return pltpu.make_async_copy(stage.at[b], acc.at[idxb.at[slot, 1]],
                                     ssem.at[b])

    for j in range(NIDX - 1):
        idx_load(j, j).start()
    rows = pl.ds(s * RPT, RPT)
    pltpu.sync_copy(hw_hbm.at[rows, cols], table.at[rows])
    for r in range(RPT // 128):
        pltpu.sync_copy(zrow_hbm, acc.at[pl.ds(s * RPT + r * 128, 128)])
    plsc.subcore_barrier()
    for j in range(LEAD):
        idx_load(j, j).wait()
        gath(j, j).start()

    def body(i, carry):
        b = lax.rem(i, NBUF)
        sl = jnp.bitwise_and(i, NIDX - 1)
        f = i + LEAD
        slf = jnp.bitwise_and(f, NIDX - 1)
        bf = lax.rem(f, NBUF)

        @pl.when(f < CPT)
        def _():
            idx_load(f, slf).wait()
            fb = f - NBUF

            @pl.when(fb >= 0)
            def _():
                scat(jnp.bitwise_and(fb, NIDX - 1), bf).wait()

            gath(slf, bf).start()
            nxt = fb + NIDX

            @pl.when((nxt >= NIDX - 1) & (nxt < CPT))
            def _():
                idx_load(nxt, jnp.bitwise_and(nxt, NIDX - 1)).start()

        gath(sl, b).wait()
        scat(sl, b).start(add=True)
        return carry

    lax.fori_loop(0, CPT, body, 0)
    for g in range(CPT - NBUF, CPT):
        scat(g & (NIDX - 1), g % NBUF).wait()
    plsc.subcore_barrier()
    pltpu.sync_copy(acc.at[rows], out_hbm.at[rows, cols])


_edge_call = pl.kernel(
    _edge_body,
    out_type=jax.ShapeDtypeStruct((NP, D), _f32),
    mesh=_mesh(),
    compiler_params=pltpu.CompilerParams(use_tc_tiling_on_sc=False),
    scratch_types=[
        pltpu.VMEM((NIDX, 2, CH), jnp.int32),
        pltpu.VMEM((NBUF, CH, DH), _f32),
        pltpu.VMEM_SHARED((NP, DH), _f32),
        pltpu.VMEM_SHARED((NP, DH), _f32),
        pltpu.SemaphoreType.DMA((NIDX,)),
        pltpu.SemaphoreType.DMA((NBUF,)),
        pltpu.SemaphoreType.DMA((NBUF,)),
    ],
)


def _mm_raw_body(x_ref, w_ref, o_ref):
    o_ref[...] = jnp.dot(x_ref[...], w_ref[...], preferred_element_type=_f32)


def _mm_raw(xp, w1):
    return pl.pallas_call(
        _mm_raw_body,
        grid=(NB,),
        in_specs=[
            pl.BlockSpec((R, D), lambda i: (i, 0)),
            pl.BlockSpec((D, H), lambda i: (0, 0)),
        ],
        out_specs=pl.BlockSpec((R, H), lambda i: (i, 0)),
        out_shape=jax.ShapeDtypeStruct((NP, H), _f32),
    )(xp, w1)


def _scale_body(xw_ref, d_ref, o_ref, t_ref):
    dinv = lax.rsqrt(d_ref[0:1, :] + d_ref[1:2, :] + 1.0)
    t = jnp.transpose(dinv, (1, 0))
    t_ref[...] = jnp.broadcast_to(t, (R, D))
    o_ref[...] = xw_ref[...] * t


def _mm_scale(xw, degs):
    return pl.pallas_call(
        _scale_body,
        grid=(NB,),
        in_specs=[
            pl.BlockSpec((R, D), lambda i: (i, 0)),
            pl.BlockSpec((NC, R), lambda i: (0, i)),
        ],
        out_specs=[
            pl.BlockSpec((R, H), lambda i: (i, 0)),
            pl.BlockSpec((R, D), lambda i: (i, 0)),
        ],
        out_shape=[
            jax.ShapeDtypeStruct((NP, H), _f32),
            jax.ShapeDtypeStruct((NP, D), _f32),
        ],
    )(xw, degs)


def _layer_body(p_ref, hw_ref, t_ref, w_ref, b_ref, o_ref):
    t = t_ref[...]
    h = jnp.maximum((p_ref[...] + hw_ref[...]) * t + b_ref[...], 0.0)
    o_ref[...] = jnp.dot(h, w_ref[...], preferred_element_type=_f32) * t


def _layer_fuse(parts, hw, tinv, w2, b1):
    return pl.pallas_call(
        _layer_body,
        grid=(NB,),
        in_specs=[
            pl.BlockSpec((R, D), lambda i: (i, 0)),
            pl.BlockSpec((R, D), lambda i: (i, 0)),
            pl.BlockSpec((R, D), lambda i: (i, 0)),
            pl.BlockSpec((D, H), lambda i: (0, 0)),
            pl.BlockSpec((1, H), lambda i: (0, 0)),
        ],
        out_specs=pl.BlockSpec((R, H), lambda i: (i, 0)),
        out_shape=jax.ShapeDtypeStruct((NP, H), _f32),
    )(parts, hw, tinv, w2, b1)


def _pool_body(p_ref, hw_ref, t_ref, b2_ref, bt_ref, u_ref,
               aw1h_ref, aw1u_ref, ab1_ref, aw2_ref, ab2_ref,
               cw1h_ref, cw1u_ref, cb1_ref, cw2_ref, cb2_ref,
               oa_ref, ov_ref, pacc, cacc):
    i = pl.program_id(0)

    @pl.when(i == 0)
    def _init():
        pacc[...] = jnp.zeros_like(pacc)
        cacc[...] = jnp.zeros_like(cacc)

    h2 = jnp.maximum((p_ref[...] + hw_ref[...]) * t_ref[...] + b2_ref[...], 0.0)
    onehot = (bt_ref[...] == lax.broadcasted_iota(jnp.int32, (G, R), 0)).astype(_f32)
    pacc[...] += jnp.dot(onehot, h2, preferred_element_type=_f32)
    cacc[...] += jnp.dot(onehot, jnp.ones((R, D), _f32), preferred_element_type=_f32)

    @pl.when(i == NB - 1)
    def _heads():
        pooled = pacc[...] / jnp.maximum(cacc[...], 1.0)
        ah = jnp.maximum(
            jnp.dot(pooled, aw1h_ref[...], preferred_element_type=_f32)
            + jnp.dot(u_ref[...], aw1u_ref[...], preferred_element_type=_f32)
            + ab1_ref[...], 0.0)
        oa_ref[...] = jnp.dot(ah, aw2_ref[...], preferred_element_type=_f32) + ab2_ref[...]
        ch = jnp.maximum(
            jnp.dot(pooled, cw1h_ref[...], preferred_element_type=_f32)
            + jnp.dot(u_ref[...], cw1u_ref[...], preferred_element_type=_f32)
            + cb1_ref[...], 0.0)
        ov_ref[...] = jnp.dot(ch, cw2_ref[...], preferred_element_type=_f32) + cb2_ref[...]


def _pool_heads(parts, hw, tinv, b2, bt, u, aW1, ab1, aW2, ab2,
                cW1, cb1, cW2, cb2):
    full = lambda shape: pl.BlockSpec(shape, lambda i: tuple(0 for _ in shape))
    return pl.pallas_call(
        _pool_body,
        grid=(NB,),
        in_specs=[
            pl.BlockSpec((R, D), lambda i: (i, 0)),
            pl.BlockSpec((R, D), lambda i: (i, 0)),
            pl.BlockSpec((R, D), lambda i: (i, 0)),
            full((1, H)),
            pl.BlockSpec((1, R), lambda i: (0, i)),
            full((G, U)),
            full((H, H)), full((U, H)), full((1, H)),
            full((H, A)), full((1, A)),
            full((H, H)), full((U, H)), full((1, H)),
            full((H, 1)), full((1, 1)),
        ],
        out_specs=[
            pl.BlockSpec((G, A), lambda i: (0, 0)),
            pl.BlockSpec((G, 1), lambda i: (0, 0)),
        ],
        out_shape=[
            jax.ShapeDtypeStruct((G, A), _f32),
            jax.ShapeDtypeStruct((G, 1), _f32),
        ],
        scratch_shapes=[
            pltpu.VMEM((G, H), _f32),
            pltpu.VMEM((G, H), _f32),
        ],
    )(parts, hw, tinv, b2, bt, u,
      aW1[:H], aW1[H:], ab1, aW2, ab2,
      cW1[:H], cW1[H:], cb1, cW2, cb2)


def kernel(x, u, W1, b1, W2, b2, aW1, ab1, aW2, ab2, cW1, cb1, cW2, cb2,
           edge_index, batch):
    xp = jnp.zeros((NP, D), _f32).at[:N].set(x)
    pad = jnp.full((2, EP - E), N, jnp.int32)
    srcdst = (jnp.concatenate([edge_index, pad], axis=1)
              .reshape(2, NS, CPT, CH).transpose(1, 2, 0, 3))
    btp = jnp.concatenate([batch, jnp.full((NP - N,), G, jnp.int32)]).reshape(1, NP)

    ones_row = jnp.ones((CH,), _f32)
    zeros_seg = jnp.zeros((RPT,), _f32)
    zeros_row = jnp.zeros((128, DH), _f32)

    degs = _deg_call(srcdst, ones_row, zeros_seg)

    xw = _mm_raw(xp, W1)
    hw1, tinv = _mm_scale(xw, degs)
    parts1 = _edge_call(hw1, srcdst, zeros_row)
    hw2 = _layer_fuse(parts1, hw1, tinv, W2, b1.reshape(1, H))
    parts2 = _edge_call(hw2, srcdst, zeros_row)

    oa, ov = _pool_heads(
        parts2, hw2, tinv, b2.reshape(1, H), btp, u,
        aW1, ab1.reshape(1, H), aW2, ab2.reshape(1, A),
        cW1, cb1.reshape(1, H), cW2, cb2.reshape(1, 1))
    return (oa, ov)


# speedup vs baseline: 1.0965x; 1.0214x over previous
import functools

import jax
import jax.numpy as jnp
from jax import lax
from jax.experimental import pallas as pl
from jax.experimental.pallas import tpu as pltpu
from jax.experimental.pallas import tpu_sc as plsc

N = 10000
E = 320000
G = 64
D = 128
H = 128
U = 32
A = 16

NP = 10240
NC = 2
NS = 16
DH = D // 2
CH = 128
CPT = 158
EP = CPT * CH * NS
CPD = CPT // 2
RPT = NP // NS
R = 2048
NB = NP // R

_f32 = jnp.float32


def _mesh():
    return plsc.VectorSubcoreMesh(
        core_axis_name="c", subcore_axis_name="s", num_cores=NC, num_subcores=NS
    )


def _deg_body(sd_hbm, ones_hbm, zeros_hbm, out_hbm, idxall, onesv, acc, sem):
    c = lax.axis_index("c")
    s = lax.axis_index("s")
    pltpu.sync_copy(zeros_hbm, acc.at[pl.ds(s * RPT, RPT)])
    pltpu.sync_copy(ones_hbm, onesv)
    pltpu.sync_copy(sd_hbm.at[s, pl.ds(c * CPD, CPD)], idxall)
    plsc.subcore_barrier()

    def fire(g, carry):
        pltpu.make_async_copy(onesv, acc.at[idxall.at[g, 1]], sem).start(add=True)
        return carry

    def drain(g, carry):
        pltpu.make_async_copy(onesv, acc.at[idxall.at[g, 1]], sem).wait()
        return carry

    lax.fori_loop(0, CPD, fire, 0)
    lax.fori_loop(0, CPD, drain, 0)
    plsc.subcore_barrier()
    pltpu.sync_copy(acc.at[pl.ds(s * RPT, RPT)], out_hbm.at[c, pl.ds(s * RPT, RPT)])


_deg_call = pl.kernel(
    _deg_body,
    out_type=jax.ShapeDtypeStruct((NC, NP), _f32),
    mesh=_mesh(),
    scratch_types=[
        pltpu.VMEM((CPD, 2, CH), jnp.int32),
        pltpu.VMEM((CH,), _f32),
        pltpu.VMEM_SHARED((NP,), _f32),
        pltpu.SemaphoreType.DMA,
    ],
)


NBUF = 5
NIDX = 8
LEAD = 3


def _edge_body(hw_hbm, sd_hbm, zrow_hbm, out_hbm, idxb, stage, table, acc,
               isem, gsem, ssem):
    c = lax.axis_index("c")
    s = lax.axis_index("s")
    cols = pl.ds(c * DH, DH)

    def idx_load(g, slot):
        return pltpu.make_async_copy(sd_hbm.at[s, g], idxb.at[slot], isem.at[slot])

    def gath(slot, b):
        return pltpu.make_async_copy(table.at[idxb.at[slot, 0]], stage.at[b],
                                     gsem.at[b])

    def scat(slot, b):
        return pltpu.make_async_copy(stage.at[b], acc.at[idxb.at[slot, 1]],
                                     ssem.at[b])

    for j in range(NIDX - 1):
        idx_load(j, j).start()
    rows = pl.ds(s * RPT, RPT)
    pltpu.sync_copy(hw_hbm.at[rows, cols], table.at[rows])
    for r in range(RPT // 128):
        pltpu.sync_copy(zrow_hbm, acc.at[pl.ds(s * RPT + r * 128, 128)])
    plsc.subcore_barrier()
    for j in range(LEAD):
        idx_load(j, j).wait()
        gath(j, j).start()

    def body(i, carry):
        b = lax.rem(i, NBUF)
        sl = jnp.bitwise_and(i, NIDX - 1)
        f = i + LEAD
        slf = jnp.bitwise_and(f, NIDX - 1)
        bf = lax.rem(f, NBUF)

        @pl.when(f < CPT)
        def _():
            idx_load(f, slf).wait()
            fb = f - NBUF

            @pl.when(fb >= 0)
            def _():
                scat(jnp.bitwise_and(fb, NIDX - 1), bf).wait()

            gath(slf, bf).start()
            nxt = fb + NIDX

            @pl.when((nxt >= NIDX - 1) & (nxt < CPT))
            def _():
                idx_load(nxt, jnp.bitwise_and(nxt, NIDX - 1)).start()

        gath(sl, b).wait()
        scat(sl, b).start(add=True)
        return carry

    lax.fori_loop(0, CPT, body, 0)
    for g in range(CPT - NBUF, CPT):
        scat(g & (NIDX - 1), g % NBUF).wait()
    plsc.subcore_barrier()
    pltpu.sync_copy(acc.at[rows], out_hbm.at[rows, cols])


_edge_call = pl.kernel(
    _edge_body,
    out_type=jax.ShapeDtypeStruct((NP, D), _f32),
    mesh=_mesh(),
    compiler_params=pltpu.CompilerParams(use_tc_tiling_on_sc=False),
    scratch_types=[
        pltpu.VMEM((NIDX, 2, CH), jnp.int32),
        pltpu.VMEM((NBUF, CH, DH), _f32),
        pltpu.VMEM_SHARED((NP, DH), _f32),
        pltpu.VMEM_SHARED((NP, DH), _f32),
        pltpu.SemaphoreType.DMA((NIDX,)),
        pltpu.SemaphoreType.DMA((NBUF,)),
        pltpu.SemaphoreType.DMA((NBUF,)),
    ],
)


def _mm_raw_body(x_ref, w_ref, o_ref):
    o_ref[...] = jnp.dot(x_ref[...], w_ref[...], preferred_element_type=_f32)


def _mm_raw(xp, w1):
    return pl.pallas_call(
        _mm_raw_body,
        grid=(NB,),
        in_specs=[
            pl.BlockSpec((R, D), lambda i: (i, 0)),
            pl.BlockSpec((D, H), lambda i: (0, 0)),
        ],
        out_specs=pl.BlockSpec((R, H), lambda i: (i, 0)),
        out_shape=jax.ShapeDtypeStruct((NP, H), _f32),
    )(xp, w1)


def _scale_body(xw_ref, d_ref, o_ref, t_ref):
    dinv = lax.rsqrt(d_ref[0:1, :] + d_ref[1:2, :] + 1.0)
    t = jnp.transpose(dinv, (1, 0))
    t_ref[...] = jnp.broadcast_to(t, (R, D))
    o_ref[...] = xw_ref[...] * t


def _mm_scale(xw, degs):
    return pl.pallas_call(
        _scale_body,
        grid=(NB,),
        in_specs=[
            pl.BlockSpec((R, D), lambda i: (i, 0)),
            pl.BlockSpec((NC, R), lambda i: (0, i)),
        ],
        out_specs=[
            pl.BlockSpec((R, H), lambda i: (i, 0)),
            pl.BlockSpec((R, D), lambda i: (i, 0)),
        ],
        out_shape=[
            jax.ShapeDtypeStruct((NP, H), _f32),
            jax.ShapeDtypeStruct((NP, D), _f32),
        ],
    )(xw, degs)


def _layer_body(p_ref, hw_ref, t_ref, w_ref, b_ref, o_ref):
    t = t_ref[...]
    h = jnp.maximum((p_ref[...] + hw_ref[...]) * t + b_ref[...], 0.0)
    o_ref[...] = jnp.dot(h, w_ref[...], preferred_element_type=_f32) * t


def _layer_fuse(parts, hw, tinv, w2, b1):
    return pl.pallas_call(
        _layer_body,
        grid=(NB,),
        in_specs=[
            pl.BlockSpec((R, D), lambda i: (i, 0)),
            pl.BlockSpec((R, D), lambda i: (i, 0)),
            pl.BlockSpec((R, D), lambda i: (i, 0)),
            pl.BlockSpec((D, H), lambda i: (0, 0)),
            pl.BlockSpec((1, H), lambda i: (0, 0)),
        ],
        out_specs=pl.BlockSpec((R, H), lambda i: (i, 0)),
        out_shape=jax.ShapeDtypeStruct((NP, H), _f32),
    )(parts, hw, tinv, w2, b1)


def _pool_body(p_ref, hw_ref, t_ref, b2_ref, bt_ref, u_ref,
               aw1h_ref, aw1u_ref, ab1_ref, aw2_ref, ab2_ref,
               cw1h_ref, cw1u_ref, cb1_ref, cw2_ref, cb2_ref,
               oa_ref, ov_ref, pacc, cacc):
    i = pl.program_id(0)

    @pl.when(i == 0)
    def _init():
        pacc[...] = jnp.zeros_like(pacc)
        cacc[...] = jnp.zeros_like(cacc)

    h2 = jnp.maximum((p_ref[...] + hw_ref[...]) * t_ref[...] + b2_ref[...], 0.0)
    onehot = (bt_ref[...] == lax.broadcasted_iota(jnp.int32, (G, R), 0)).astype(_f32)
    pacc[...] += jnp.dot(onehot, h2, preferred_element_type=_f32)
    cacc[...] += jnp.dot(onehot, jnp.ones((R, D), _f32), preferred_element_type=_f32)

    @pl.when(i == NB - 1)
    def _heads():
        pooled = pacc[...] / jnp.maximum(cacc[...], 1.0)
        ah = jnp.maximum(
            jnp.dot(pooled, aw1h_ref[...], preferred_element_type=_f32)
            + jnp.dot(u_ref[...], aw1u_ref[...], preferred_element_type=_f32)
            + ab1_ref[...], 0.0)
        oa_ref[...] = jnp.dot(ah, aw2_ref[...], preferred_element_type=_f32) + ab2_ref[...]
        ch = jnp.maximum(
            jnp.dot(pooled, cw1h_ref[...], preferred_element_type=_f32)
            + jnp.dot(u_ref[...], cw1u_ref[...], preferred_element_type=_f32)
            + cb1_ref[...], 0.0)
        ov_ref[...] = jnp.dot(ch, cw2_ref[...], preferred_element_type=_f32) + cb2_ref[...]


def _pool_heads(parts, hw, tinv, b2, bt, u, aW1, ab1, aW2, ab2,
                cW1, cb1, cW2, cb2):
    full = lambda shape: pl.BlockSpec(shape, lambda i: tuple(0 for _ in shape))
    return pl.pallas_call(
        _pool_body,
        grid=(NB,),
        in_specs=[
            pl.BlockSpec((R, D), lambda i: (i, 0)),
            pl.BlockSpec((R, D), lambda i: (i, 0)),
            pl.BlockSpec((R, D), lambda i: (i, 0)),
            full((1, H)),
            pl.BlockSpec((1, R), lambda i: (0, i)),
            full((G, U)),
            full((H, H)), full((U, H)), full((1, H)),
            full((H, A)), full((1, A)),
            full((H, H)), full((U, H)), full((1, H)),
            full((H, 1)), full((1, 1)),
        ],
        out_specs=[
            pl.BlockSpec((G, A), lambda i: (0, 0)),
            pl.BlockSpec((G, 1), lambda i: (0, 0)),
        ],
        out_shape=[
            jax.ShapeDtypeStruct((G, A), _f32),
            jax.ShapeDtypeStruct((G, 1), _f32),
        ],
        scratch_shapes=[
            pltpu.VMEM((G, H), _f32),
            pltpu.VMEM((G, H), _f32),
        ],
    )(parts, hw, tinv, b2, bt, u,
      aW1[:H], aW1[H:], ab1, aW2, ab2,
      cW1[:H], cW1[H:], cb1, cW2, cb2)


def kernel(x, u, W1, b1, W2, b2, aW1, ab1, aW2, ab2, cW1, cb1, cW2, cb2,
           edge_index, batch):
    xp = jnp.zeros((NP, D), _f32).at[:N].set(x)
    pad = jnp.full((2, EP - E), N, jnp.int32)
    srcdst = (jnp.concatenate([edge_index, pad], axis=1)
              .reshape(2, NS, CPT, CH).transpose(1, 2, 0, 3))
    btp = jnp.concatenate([batch, jnp.full((NP - N,), G, jnp.int32)]).reshape(1, NP)

    ones_row = jnp.ones((CH,), _f32)
    zeros_seg = jnp.zeros((RPT,), _f32)
    zeros_row = jnp.zeros((128, DH), _f32)

    degs = _deg_call(srcdst, ones_row, zeros_seg)

    xw = _mm_raw(xp, W1)
    hw1, tinv = _mm_scale(xw, degs)
    parts1 = _edge_call(hw1, srcdst, zeros_row)
    hw2 = _layer_fuse(parts1, hw1, tinv, W2, b1.reshape(1, H))
    parts2 = _edge_call(hw2, srcdst, zeros_row)

    oa, ov = _pool_heads(
        parts2, hw2, tinv, b2.reshape(1, H), btp, u,
        aW1, ab1.reshape(1, H), aW2, ab2.reshape(1, A),
        cW1, cb1.reshape(1, H), cW2, cb2.reshape(1, 1))
    return (oa, ov)


# speedup vs baseline: 1.1117x; 1.0138x over previous
import functools

import jax
import jax.numpy as jnp
from jax import lax
from jax.experimental import pallas as pl
from jax.experimental.pallas import tpu as pltpu
from jax.experimental.pallas import tpu_sc as plsc

N = 10000
E = 320000
G = 64
D = 128
H = 128
U = 32
A = 16

NP = 10240
NC = 2
NS = 16
DH = D // 2
CH = 128
CPT = 158
EP = CPT * CH * NS
CPD = CPT // 2
RPT = NP // NS
R = 5120
NB = NP // R

_f32 = jnp.float32


def _mesh():
    return plsc.VectorSubcoreMesh(
        core_axis_name="c", subcore_axis_name="s", num_cores=NC, num_subcores=NS
    )


def _deg_body(sd_hbm, ones_hbm, zeros_hbm, out_hbm, idxall, onesv, acc, sem):
    c = lax.axis_index("c")
    s = lax.axis_index("s")
    pltpu.sync_copy(zeros_hbm, acc.at[pl.ds(s * RPT, RPT)])
    pltpu.sync_copy(ones_hbm, onesv)
    pltpu.sync_copy(sd_hbm.at[s, pl.ds(c * CPD, CPD)], idxall)
    plsc.subcore_barrier()

    def fire(g, carry):
        pltpu.make_async_copy(onesv, acc.at[idxall.at[g, 1]], sem).start(add=True)
        return carry

    def drain(g, carry):
        pltpu.make_async_copy(onesv, acc.at[idxall.at[g, 1]], sem).wait()
        return carry

    lax.fori_loop(0, CPD, fire, 0)
    lax.fori_loop(0, CPD, drain, 0)
    plsc.subcore_barrier()
    pltpu.sync_copy(acc.at[pl.ds(s * RPT, RPT)], out_hbm.at[c, pl.ds(s * RPT, RPT)])


_deg_call = pl.kernel(
    _deg_body,
    out_type=jax.ShapeDtypeStruct((NC, NP), _f32),
    mesh=_mesh(),
    scratch_types=[
        pltpu.VMEM((CPD, 2, CH), jnp.int32),
        pltpu.VMEM((CH,), _f32),
        pltpu.VMEM_SHARED((NP,), _f32),
        pltpu.SemaphoreType.DMA,
    ],
)


NBUF = 5
NIDX = 8
LEAD = 3


def _edge_body(hw_hbm, sd_hbm, zrow_hbm, out_hbm, idxb, stage, table, acc,
               isem, gsem, ssem):
    c = lax.axis_index("c")
    s = lax.axis_index("s")
    cols = pl.ds(c * DH, DH)

    def idx_load(g, slot):
        return pltpu.make_async_copy(sd_hbm.at[s, g], idxb.at[slot], isem.at[slot])

    def gath(slot, b):
        return pltpu.make_async_copy(table.at[idxb.at[slot, 0]], stage.at[b],
                                     gsem.at[b])

    def scat(slot, b):
        return pltpu.make_async_copy(stage.at[b], acc.at[idxb.at[slot, 1]],
                                     ssem.at[b])

    for j in range(NIDX - 1):
        idx_load(j, j).start()
    rows = pl.ds(s * RPT, RPT)
    pltpu.sync_copy(hw_hbm.at[rows, cols], table.at[rows])
    for r in range(RPT // 128):
        pltpu.sync_copy(zrow_hbm, acc.at[pl.ds(s * RPT + r * 128, 128)])
    plsc.subcore_barrier()
    for j in range(LEAD):
        idx_load(j, j).wait()
        gath(j, j).start()

    def body(i, carry):
        b = lax.rem(i, NBUF)
        sl = jnp.bitwise_and(i, NIDX - 1)
        f = i + LEAD
        slf = jnp.bitwise_and(f, NIDX - 1)
        bf = lax.rem(f, NBUF)

        @pl.when(f < CPT)
        def _():
            idx_load(f, slf).wait()
            fb = f - NBUF

            @pl.when(fb >= 0)
            def _():
                scat(jnp.bitwise_and(fb, NIDX - 1), bf).wait()

            gath(slf, bf).start()
            nxt = fb + NIDX

            @pl.when((nxt >= NIDX - 1) & (nxt < CPT))
            def _():
                idx_load(nxt, jnp.bitwise_and(nxt, NIDX - 1)).start()

        gath(sl, b).wait()
        scat(sl, b).start(add=True)
        return carry

    lax.fori_loop(0, CPT, body, 0)
    for g in range(CPT - NBUF, CPT):
        scat(g & (NIDX - 1), g % NBUF).wait()
    plsc.subcore_barrier()
    pltpu.sync_copy(acc.at[rows], out_hbm.at[rows, cols])


_edge_call = pl.kernel(
    _edge_body,
    out_type=jax.ShapeDtypeStruct((NP, D), _f32),
    mesh=_mesh(),
    compiler_params=pltpu.CompilerParams(use_tc_tiling_on_sc=False),
    scratch_types=[
        pltpu.VMEM((NIDX, 2, CH), jnp.int32),
        pltpu.VMEM((NBUF, CH, DH), _f32),
        pltpu.VMEM_SHARED((NP, DH), _f32),
        pltpu.VMEM_SHARED((NP, DH), _f32),
        pltpu.SemaphoreType.DMA((NIDX,)),
        pltpu.SemaphoreType.DMA((NBUF,)),
        pltpu.SemaphoreType.DMA((NBUF,)),
    ],
)


def _mm_raw_body(x_ref, w_ref, o_ref):
    o_ref[...] = jnp.dot(x_ref[...], w_ref[...], preferred_element_type=_f32)


def _mm_raw(xp, w1):
    return pl.pallas_call(
        _mm_raw_body,
        grid=(NB,),
        in_specs=[
            pl.BlockSpec((R, D), lambda i: (i, 0)),
            pl.BlockSpec((D, H), lambda i: (0, 0)),
        ],
        out_specs=pl.BlockSpec((R, H), lambda i: (i, 0)),
        out_shape=jax.ShapeDtypeStruct((NP, H), _f32),
    )(xp, w1)


def _scale_body(xw_ref, d_ref, o_ref, t_ref):
    dinv = lax.rsqrt(d_ref[0:1, :] + d_ref[1:2, :] + 1.0)
    t = jnp.transpose(dinv, (1, 0))
    t_ref[...] = jnp.broadcast_to(t, (R, D))
    o_ref[...] = xw_ref[...] * t


def _mm_scale(xw, degs):
    return pl.pallas_call(
        _scale_body,
        grid=(NB,),
        in_specs=[
            pl.BlockSpec((R, D), lambda i: (i, 0)),
            pl.BlockSpec((NC, R), lambda i: (0, i)),
        ],
        out_specs=[
            pl.BlockSpec((R, H), lambda i: (i, 0)),
            pl.BlockSpec((R, D), lambda i: (i, 0)),
        ],
        out_shape=[
            jax.ShapeDtypeStruct((NP, H), _f32),
            jax.ShapeDtypeStruct((NP, D), _f32),
        ],
    )(xw, degs)


def _layer_body(p_ref, hw_ref, t_ref, w_ref, b_ref, o_ref):
    t = t_ref[...]
    h = jnp.maximum((p_ref[...] + hw_ref[...]) * t + b_ref[...], 0.0)
    o_ref[...] = jnp.dot(h, w_ref[...], preferred_element_type=_f32) * t


def _layer_fuse(parts, hw, tinv, w2, b1):
    return pl.pallas_call(
        _layer_body,
        grid=(NB,),
        in_specs=[
            pl.BlockSpec((R, D), lambda i: (i, 0)),
            pl.BlockSpec((R, D), lambda i: (i, 0)),
            pl.BlockSpec((R, D), lambda i: (i, 0)),
            pl.BlockSpec((D, H), lambda i: (0, 0)),
            pl.BlockSpec((1, H), lambda i: (0, 0)),
        ],
        out_specs=pl.BlockSpec((R, H), lambda i: (i, 0)),
        out_shape=jax.ShapeDtypeStruct((NP, H), _f32),
    )(parts, hw, tinv, w2, b1)


def _pool_body(p_ref, hw_ref, t_ref, b2_ref, bt_ref, u_ref,
               aw1h_ref, aw1u_ref, ab1_ref, aw2_ref, ab2_ref,
               cw1h_ref, cw1u_ref, cb1_ref, cw2_ref, cb2_ref,
               oa_ref, ov_ref, pacc, cacc):
    i = pl.program_id(0)

    @pl.when(i == 0)
    def _init():
        pacc[...] = jnp.zeros_like(pacc)
        cacc[...] = jnp.zeros_like(cacc)

    h2 = jnp.maximum((p_ref[...] + hw_ref[...]) * t_ref[...] + b2_ref[...], 0.0)
    onehot = (bt_ref[...] == lax.broadcasted_iota(jnp.int32, (G, R), 0)).astype(_f32)
    pacc[...] += jnp.dot(onehot, h2, preferred_element_type=_f32)
    cacc[...] += jnp.dot(onehot, jnp.ones((R, D), _f32), preferred_element_type=_f32)

    @pl.when(i == NB - 1)
    def _heads():
        pooled = pacc[...] / jnp.maximum(cacc[...], 1.0)
        ah = jnp.maximum(
            jnp.dot(pooled, aw1h_ref[...], preferred_element_type=_f32)
            + jnp.dot(u_ref[...], aw1u_ref[...], preferred_element_type=_f32)
            + ab1_ref[...], 0.0)
        oa_ref[...] = jnp.dot(ah, aw2_ref[...], preferred_element_type=_f32) + ab2_ref[...]
        ch = jnp.maximum(
            jnp.dot(pooled, cw1h_ref[...], preferred_element_type=_f32)
            + jnp.dot(u_ref[...], cw1u_ref[...], preferred_element_type=_f32)
            + cb1_ref[...], 0.0)
        ov_ref[...] = jnp.dot(ch, cw2_ref[...], preferred_element_type=_f32) + cb2_ref[...]


def _pool_heads(parts, hw, tinv, b2, bt, u, aW1, ab1, aW2, ab2,
                cW1, cb1, cW2, cb2):
    full = lambda shape: pl.BlockSpec(shape, lambda i: tuple(0 for _ in shape))
    return pl.pallas_call(
        _pool_body,
        grid=(NB,),
        in_specs=[
            pl.BlockSpec((R, D), lambda i: (i, 0)),
            pl.BlockSpec((R, D), lambda i: (i, 0)),
            pl.BlockSpec((R, D), lambda i: (i, 0)),
            full((1, H)),
            pl.BlockSpec((1, R), lambda i: (0, i)),
            full((G, U)),
            full((H, H)), full((U, H)), full((1, H)),
            full((H, A)), full((1, A)),
            full((H, H)), full((U, H)), full((1, H)),
            full((H, 1)), full((1, 1)),
        ],
        out_specs=[
            pl.BlockSpec((G, A), lambda i: (0, 0)),
            pl.BlockSpec((G, 1), lambda i: (0, 0)),
        ],
        out_shape=[
            jax.ShapeDtypeStruct((G, A), _f32),
            jax.ShapeDtypeStruct((G, 1), _f32),
        ],
        scratch_shapes=[
            pltpu.VMEM((G, H), _f32),
            pltpu.VMEM((G, H), _f32),
        ],
    )(parts, hw, tinv, b2, bt, u,
      aW1[:H], aW1[H:], ab1, aW2, ab2,
      cW1[:H], cW1[H:], cb1, cW2, cb2)


def kernel(x, u, W1, b1, W2, b2, aW1, ab1, aW2, ab2, cW1, cb1, cW2, cb2,
           edge_index, batch):
    xp = jnp.zeros((NP, D), _f32).at[:N].set(x)
    pad = jnp.full((2, EP - E), N, jnp.int32)
    srcdst = (jnp.concatenate([edge_index, pad], axis=1)
              .reshape(2, NS, CPT, CH).transpose(1, 2, 0, 3))
    btp = jnp.concatenate([batch, jnp.full((NP - N,), G, jnp.int32)]).reshape(1, NP)

    ones_row = jnp.ones((CH,), _f32)
    zeros_seg = jnp.zeros((RPT,), _f32)
    zeros_row = jnp.zeros((128, DH), _f32)

    degs = _deg_call(srcdst, ones_row, zeros_seg)

    xw = _mm_raw(xp, W1)
    hw1, tinv = _mm_scale(xw, degs)
    parts1 = _edge_call(hw1, srcdst, zeros_row)
    hw2 = _layer_fuse(parts1, hw1, tinv, W2, b1.reshape(1, H))
    parts2 = _edge_call(hw2, srcdst, zeros_row)

    oa, ov = _pool_heads(
        parts2, hw2, tinv, b2.reshape(1, H), btp, u,
        aW1, ab1.reshape(1, H), aW2, ab2.reshape(1, A),
        cW1, cb1.reshape(1, H), cW2, cb2.reshape(1, 1))
    return (oa, ov)
